# baseline (speedup 1.0000x reference)
import jax
import jax.numpy as jnp
from jax.experimental import pallas as pl

NUM_PATCHES = 64
PATCH_AREA = 108
PROJ_DIM = 128

BLOCK_ROWS = 4096


def _patch_encoder_kernel(x_ref, w_ref, pb_ref, o_ref):
    y = jax.lax.dot_general(
        x_ref[...], w_ref[...],
        dimension_numbers=(((1,), (0,)), ((), ())),
        preferred_element_type=jnp.float32,
    )
    o_ref[...] = (y.reshape(-1, NUM_PATCHES, PROJ_DIM) + pb_ref[...]).reshape(
        -1, PROJ_DIM
    )


@jax.jit
def kernel(patches, W, b, pos_table):
    batch = patches.shape[0]
    rows = batch * NUM_PATCHES
    x = patches.reshape(rows, PATCH_AREA)
    pb = pos_table + b[None, :]
    grid = (rows // BLOCK_ROWS,)
    out = pl.pallas_call(
        _patch_encoder_kernel,
        grid=grid,
        in_specs=[
            pl.BlockSpec((BLOCK_ROWS, PATCH_AREA), lambda i: (i, 0)),
            pl.BlockSpec((PATCH_AREA, PROJ_DIM), lambda i: (0, 0)),
            pl.BlockSpec((NUM_PATCHES, PROJ_DIM), lambda i: (0, 0)),
        ],
        out_specs=pl.BlockSpec((BLOCK_ROWS, PROJ_DIM), lambda i: (i, 0)),
        out_shape=jax.ShapeDtypeStruct((rows, PROJ_DIM), jnp.float32),
    )(x, W, pb)
    return out.reshape(batch, NUM_PATCHES, PROJ_DIM)


# speedup vs baseline: 1.0886x; 1.0886x over previous
import jax
import jax.numpy as jnp
from jax.experimental import pallas as pl

NUM_PATCHES = 64
PATCH_AREA = 108
PROJ_DIM = 128

BLOCK_ROWS = 16384


def _patch_encoder_kernel(x_ref, w_ref, pb_ref, o_ref):
    y = jax.lax.dot_general(
        x_ref[...], w_ref[...],
        dimension_numbers=(((1,), (0,)), ((), ())),
        preferred_element_type=jnp.float32,
    )
    o_ref[...] = (y.reshape(-1, NUM_PATCHES, PROJ_DIM) + pb_ref[...]).reshape(
        -1, PROJ_DIM
    )


@jax.jit
def kernel(patches, W, b, pos_table):
    batch = patches.shape[0]
    rows = batch * NUM_PATCHES
    x = patches.reshape(rows, PATCH_AREA)
    pb = pos_table + b[None, :]
    grid = (rows // BLOCK_ROWS,)
    out = pl.pallas_call(
        _patch_encoder_kernel,
        grid=grid,
        in_specs=[
            pl.BlockSpec((BLOCK_ROWS, PATCH_AREA), lambda i: (i, 0)),
            pl.BlockSpec((PATCH_AREA, PROJ_DIM), lambda i: (0, 0)),
            pl.BlockSpec((NUM_PATCHES, PROJ_DIM), lambda i: (0, 0)),
        ],
        out_specs=pl.BlockSpec((BLOCK_ROWS, PROJ_DIM), lambda i: (i, 0)),
        out_shape=jax.ShapeDtypeStruct((rows, PROJ_DIM), jnp.float32),
    )(x, W, pb)
    return out.reshape(batch, NUM_PATCHES, PROJ_DIM)
